# Initial kernel scaffold; baseline (speedup 1.0000x reference)
import jax
import jax.numpy as jnp
from jax.experimental import pallas as pl


def kernel(x, weight, bias):
    raise NotImplementedError("write your pallas kernel here")


# speedup vs baseline: 2.7527x; 2.7527x over previous
import jax
import jax.numpy as jnp
from jax.experimental import pallas as pl
from jax.experimental.pallas import tpu as pltpu


def _linear_kernel(x_ref, w_ref, b_ref, o_ref):
    acc = jax.lax.dot_general(
        x_ref[...], w_ref[...],
        dimension_numbers=(((1,), (1,)), ((), ())),
        preferred_element_type=jnp.float32,
    )
    o_ref[...] = acc + b_ref[...]


def kernel(x, weight, bias):
    B, K = x.shape
    N, _ = weight.shape
    tm = 512
    tn = 1024
    gm = B // tm
    gn = N // tn

    b2 = bias.reshape(1, N)

    return pl.pallas_call(
        _linear_kernel,
        grid=(gn, gm),
        in_specs=[
            pl.BlockSpec((tm, K), lambda j, i: (i, 0)),
            pl.BlockSpec((tn, K), lambda j, i: (j, 0)),
            pl.BlockSpec((1, tn), lambda j, i: (0, j)),
        ],
        out_specs=pl.BlockSpec((tm, tn), lambda j, i: (i, j)),
        out_shape=jax.ShapeDtypeStruct((B, N), x.dtype),
        compiler_params=pltpu.CompilerParams(
            dimension_semantics=("arbitrary", "arbitrary"),
            vmem_limit_bytes=60000 * 1024,
        ),
        cost_estimate=pl.CostEstimate(
            flops=2 * B * K * N, transcendentals=0,
            bytes_accessed=B * K * 4 + N * K * 4 + B * N * 4,
        ),
    )(x, weight, b2)
